# baseline (speedup 1.0000x reference)
import jax
import jax.numpy as jnp
from jax.experimental import pallas as pl

C = 64
K = 8
D_IN = 64
D_OUT = 64
CP = C * D_OUT


def _fused_body(x_ref, ctrs_ref, wcat_ref, rcat_ref, ov_ref, out_ref):
    x = x_ref[...]
    ctrs = ctrs_ref[...]
    T = x.shape[0]

    xc = jax.lax.dot_general(
        x, ctrs, dimension_numbers=(((1,), (1,)), ((), ())),
        precision=jax.lax.Precision.HIGHEST,
        preferred_element_type=jnp.float32)
    c_sq = jnp.sum(ctrs * ctrs, axis=1)[None, :]
    d = c_sq - 2.0 * xc

    work = d
    m0 = None
    tk = None
    for i in range(K):
        tk = jnp.min(work, axis=1, keepdims=True)
        if i == 0:
            m0 = tk
        if i < K - 1:
            work = jnp.where(work == tk, jnp.float32(jnp.inf), work)

    e = jnp.where(d <= tk, jnp.exp(m0 - d), 0.0)
    a = e / jnp.sum(e, axis=1, keepdims=True)

    x_hi = x.astype(jnp.bfloat16)
    x_lo = (x - x_hi.astype(jnp.float32)).astype(jnp.bfloat16)
    xcat = jnp.concatenate([x_hi, x_hi, x_lo], axis=1)
    y = jax.lax.dot_general(
        xcat, wcat_ref[...],
        dimension_numbers=(((1,), (0,)), ((), ())),
        preferred_element_type=jnp.float32)

    a_hi = a.astype(jnp.bfloat16)
    arep = jax.lax.dot_general(
        a_hi, rcat_ref[...],
        dimension_numbers=(((1,), (0,)), ((), ())),
        preferred_element_type=jnp.float32)
    acc = jax.lax.dot_general(
        a_hi, ov_ref[...], dimension_numbers=(((1,), (0,)), ((), ())),
        preferred_element_type=jnp.float32)

    q = CP // 4
    z = (y[:, :q] * arep[:, :q] + y[:, q:2 * q] * arep[:, q:2 * q]) + \
        (y[:, 2 * q:3 * q] * arep[:, 2 * q:3 * q]
         + y[:, 3 * q:] * arep[:, 3 * q:])
    w = q
    while w > D_OUT:
        w //= 4
        z = (z[:, :w] + z[:, w:2 * w]) + (z[:, 2 * w:3 * w] + z[:, 3 * w:4 * w])
    out_ref[...] = acc + z


@jax.jit
def kernel(x, ctrs, Wv, Ov):
    n = x.shape[0]
    tile = 512
    grid = (n // tile,)
    f32, bf16 = jnp.float32, jnp.bfloat16

    wvt = jnp.transpose(Wv, (1, 0, 2)).reshape(D_IN, CP)
    w_hi = wvt.astype(bf16)
    w_lo = (wvt - w_hi.astype(f32)).astype(bf16)
    wcat = jnp.concatenate([w_hi, w_lo, w_hi], axis=0)

    rcat = jnp.repeat(jnp.eye(C, dtype=f32), D_OUT, axis=1).astype(bf16)
    ov_bf = Ov.astype(bf16)

    return pl.pallas_call(
        _fused_body,
        grid=grid,
        in_specs=[
            pl.BlockSpec((tile, D_IN), lambda i: (i, 0)),
            pl.BlockSpec((C, D_IN), lambda i: (0, 0)),
            pl.BlockSpec((3 * D_IN, CP), lambda i: (0, 0)),
            pl.BlockSpec((C, CP), lambda i: (0, 0)),
            pl.BlockSpec((C, D_OUT), lambda i: (0, 0)),
        ],
        out_specs=pl.BlockSpec((tile, D_OUT), lambda i: (i, 0)),
        out_shape=jax.ShapeDtypeStruct((n, D_OUT), jnp.float32),
    )(x, ctrs, wcat, rcat, ov_bf)


# speedup vs baseline: 17.4641x; 17.4641x over previous
import jax
import jax.numpy as jnp
from jax.experimental import pallas as pl

C = 64
K = 8
D_IN = 64
D_OUT = 64
CP = C * D_OUT


def _fused_body(x_ref, ctrs_ref, wcat_ref, rcat_ref, ov_ref, out_ref):
    x = x_ref[...]
    ctrs = ctrs_ref[...]
    T = x.shape[0]

    xc = jax.lax.dot_general(
        x, ctrs, dimension_numbers=(((1,), (1,)), ((), ())),
        precision=jax.lax.Precision.HIGHEST,
        preferred_element_type=jnp.float32)
    x_sq = jnp.sum(x * x, axis=1, keepdims=True)
    c_sq = jnp.sum(ctrs * ctrs, axis=1)[None, :]
    d = x_sq - 2.0 * xc + c_sq

    work = d
    mask = jnp.zeros((T, C), jnp.bool_)
    m0 = None
    for _ in range(K):
        mk = jnp.min(work, axis=1, keepdims=True)
        if m0 is None:
            m0 = mk
        sel = work == mk
        mask = jnp.logical_or(mask, sel)
        work = jnp.where(sel, jnp.float32(jnp.inf), work)

    e = jnp.where(mask, jnp.exp(-(d - m0)), 0.0)
    a = e / jnp.sum(e, axis=1, keepdims=True)

    x_hi = x.astype(jnp.bfloat16)
    x_lo = (x - x_hi.astype(jnp.float32)).astype(jnp.bfloat16)
    xcat = jnp.concatenate([x_hi, x_hi, x_lo], axis=1)
    y = jax.lax.dot_general(
        xcat, wcat_ref[...],
        dimension_numbers=(((1,), (0,)), ((), ())),
        preferred_element_type=jnp.float32)

    a_hi = a.astype(jnp.bfloat16)
    arep = jax.lax.dot_general(
        a_hi, rcat_ref[...],
        dimension_numbers=(((1,), (0,)), ((), ())),
        preferred_element_type=jnp.float32)

    q = CP // 4
    z = (y[:, :q] * arep[:, :q] + y[:, q:2 * q] * arep[:, q:2 * q]) + \
        (y[:, 2 * q:3 * q] * arep[:, 2 * q:3 * q]
         + y[:, 3 * q:] * arep[:, 3 * q:])
    w = q
    while w > D_OUT:
        w //= 4
        z = (z[:, :w] + z[:, w:2 * w]) + (z[:, 2 * w:3 * w] + z[:, 3 * w:4 * w])
    acc = jax.lax.dot_general(
        a, ov_ref[...], dimension_numbers=(((1,), (0,)), ((), ())),
        precision=jax.lax.Precision.HIGHEST,
        preferred_element_type=jnp.float32)
    out_ref[...] = acc + z


@jax.jit
def kernel(x, ctrs, Wv, Ov):
    n = x.shape[0]
    tile = 1024
    grid = (n // tile,)
    f32, bf16 = jnp.float32, jnp.bfloat16

    wvt = jnp.transpose(Wv, (1, 0, 2)).reshape(D_IN, CP)
    w_hi = wvt.astype(bf16)
    w_lo = (wvt - w_hi.astype(f32)).astype(bf16)
    wcat = jnp.concatenate([w_hi, w_lo, w_hi], axis=0)

    rcat = jnp.repeat(jnp.eye(C, dtype=f32), D_OUT, axis=1).astype(bf16)

    return pl.pallas_call(
        _fused_body,
        grid=grid,
        in_specs=[
            pl.BlockSpec((tile, D_IN), lambda i: (i, 0)),
            pl.BlockSpec((C, D_IN), lambda i: (0, 0)),
            pl.BlockSpec((3 * D_IN, CP), lambda i: (0, 0)),
            pl.BlockSpec((C, CP), lambda i: (0, 0)),
            pl.BlockSpec((C, D_OUT), lambda i: (0, 0)),
        ],
        out_specs=pl.BlockSpec((tile, D_OUT), lambda i: (i, 0)),
        out_shape=jax.ShapeDtypeStruct((n, D_OUT), jnp.float32),
    )(x, ctrs, wcat, rcat, Ov)


# speedup vs baseline: 18.6944x; 1.0704x over previous
import jax
import jax.numpy as jnp
from jax.experimental import pallas as pl

C = 64
K = 8
D_IN = 64
D_OUT = 64
CP = C * D_OUT


def _fused_body(x_ref, ctrs_ref, wcat_ref, rcat_ref, ov_ref, out_ref):
    x = x_ref[...]
    ctrs = ctrs_ref[...]
    T = x.shape[0]

    xc = jax.lax.dot_general(
        x, ctrs, dimension_numbers=(((1,), (1,)), ((), ())),
        precision=jax.lax.Precision.HIGHEST,
        preferred_element_type=jnp.float32)
    x_sq = jnp.sum(x * x, axis=1, keepdims=True)
    c_sq = jnp.sum(ctrs * ctrs, axis=1)[None, :]
    d = x_sq - 2.0 * xc + c_sq

    work = d
    m0 = None
    tk = None
    for i in range(K):
        tk = jnp.min(work, axis=1, keepdims=True)
        if i == 0:
            m0 = tk
        if i < K - 1:
            work = jnp.where(work == tk, jnp.float32(jnp.inf), work)

    e = jnp.where(d <= tk, jnp.exp(-(d - m0)), 0.0)
    a = e / jnp.sum(e, axis=1, keepdims=True)

    x_hi = x.astype(jnp.bfloat16)
    x_lo = (x - x_hi.astype(jnp.float32)).astype(jnp.bfloat16)
    xcat = jnp.concatenate([x_hi, x_hi, x_lo], axis=1)
    y = jax.lax.dot_general(
        xcat, wcat_ref[...],
        dimension_numbers=(((1,), (0,)), ((), ())),
        preferred_element_type=jnp.float32)

    a_hi = a.astype(jnp.bfloat16)
    arep = jax.lax.dot_general(
        a_hi, rcat_ref[...],
        dimension_numbers=(((1,), (0,)), ((), ())),
        preferred_element_type=jnp.float32)

    q = CP // 4
    z = (y[:, :q] * arep[:, :q] + y[:, q:2 * q] * arep[:, q:2 * q]) + \
        (y[:, 2 * q:3 * q] * arep[:, 2 * q:3 * q]
         + y[:, 3 * q:] * arep[:, 3 * q:])
    w = q
    while w > D_OUT:
        w //= 4
        z = (z[:, :w] + z[:, w:2 * w]) + (z[:, 2 * w:3 * w] + z[:, 3 * w:4 * w])
    acc = jax.lax.dot_general(
        a, ov_ref[...], dimension_numbers=(((1,), (0,)), ((), ())),
        precision=jax.lax.Precision.HIGHEST,
        preferred_element_type=jnp.float32)
    out_ref[...] = acc + z


@jax.jit
def kernel(x, ctrs, Wv, Ov):
    n = x.shape[0]
    tile = 1024
    grid = (n // tile,)
    f32, bf16 = jnp.float32, jnp.bfloat16

    wvt = jnp.transpose(Wv, (1, 0, 2)).reshape(D_IN, CP)
    w_hi = wvt.astype(bf16)
    w_lo = (wvt - w_hi.astype(f32)).astype(bf16)
    wcat = jnp.concatenate([w_hi, w_lo, w_hi], axis=0)

    rcat = jnp.repeat(jnp.eye(C, dtype=f32), D_OUT, axis=1).astype(bf16)

    return pl.pallas_call(
        _fused_body,
        grid=grid,
        in_specs=[
            pl.BlockSpec((tile, D_IN), lambda i: (i, 0)),
            pl.BlockSpec((C, D_IN), lambda i: (0, 0)),
            pl.BlockSpec((3 * D_IN, CP), lambda i: (0, 0)),
            pl.BlockSpec((C, CP), lambda i: (0, 0)),
            pl.BlockSpec((C, D_OUT), lambda i: (0, 0)),
        ],
        out_specs=pl.BlockSpec((tile, D_OUT), lambda i: (i, 0)),
        out_shape=jax.ShapeDtypeStruct((n, D_OUT), jnp.float32),
    )(x, ctrs, wcat, rcat, Ov)
